# baseline (speedup 1.0000x reference)
import functools

import jax
import jax.numpy as jnp
from jax import lax
from jax.experimental import pallas as pl
from jax.experimental.pallas import tpu as pltpu
from jax.experimental.pallas import tpu_sc as plsc

NUM_USERS = 1000000
NUM_MOVIES = 100000
NUM_CLUSTERS = 1000
EMB = 64
BATCH = 16384

NC, NS = 2, 16
NW = NC * NS

NSPLIT = 2
HB = BATCH // NSPLIT
BPW = HB // NW

BBLK = 2048


def _gather_body(ut, mt, ct, ui, mi, ci, ou, om, oc, idx_v, rows_v, sem):
    wid = lax.axis_index("s") * NC + lax.axis_index("c")
    base = wid * BPW
    for tbl, idxr, outr in ((ut, ui, ou), (mt, mi, om), (ct, ci, oc)):
        pltpu.sync_copy(idxr.at[pl.ds(base, BPW)], idx_v)

        def body(g, _):
            v = idx_v[pl.ds(g * 16, 16)]
            for l in range(16):
                idx = v[l]
                pltpu.async_copy(tbl.at[idx >> 3, idx & 7],
                                 rows_v.at[g * 16 + l], sem)
            return 0

        lax.fori_loop(0, BPW // 16, body, 0)
        pltpu.make_async_copy(outr.at[pl.ds(base, BPW)], rows_v, sem).wait()
        pltpu.sync_copy(rows_v, outr.at[pl.ds(base, BPW)])


@functools.lru_cache(maxsize=None)
def _make_gather():
    return pl.kernel(
        _gather_body,
        out_type=(
            jax.ShapeDtypeStruct((HB, EMB), jnp.float32),
            jax.ShapeDtypeStruct((HB, EMB), jnp.float32),
            jax.ShapeDtypeStruct((HB, EMB), jnp.float32),
        ),
        mesh=plsc.VectorSubcoreMesh(core_axis_name="c", subcore_axis_name="s",
                                    num_cores=NC, num_subcores=NS),
        scratch_types=[
            pltpu.VMEM((BPW,), jnp.int32),
            pltpu.VMEM((BPW, EMB), jnp.float32),
            pltpu.SemaphoreType.DMA,
        ],
        compiler_params=pltpu.CompilerParams(needs_layout_passes=False),
    )


def _mlp_body(u, m, c, an, w1u, w1m, w1c, w1an, b1, w2, b2, w3, b3, out):
    h = jnp.dot(u[:], w1u[:], preferred_element_type=jnp.float32)
    h = h + jnp.dot(m[:], w1m[:], preferred_element_type=jnp.float32)
    h = h + jnp.dot(c[:], w1c[:], preferred_element_type=jnp.float32)
    h = h + jnp.dot(an[:], w1an[:], preferred_element_type=jnp.float32)
    h = jnp.maximum(h + b1[:], 0.0)
    h2 = jnp.dot(h, w2[:], preferred_element_type=jnp.float32) + b2[:]
    h2 = jnp.maximum(h2, 0.0)
    out[:] = jnp.sum(h2 * w3[:], axis=1) + b3[0, 0]


def _mlp(u, m, c, an, w1u, w1m, w1c, w1an, b1, w2, b2, w3, b3):
    nblk = HB // BBLK
    data_spec = lambda d: pl.BlockSpec((BBLK, d), lambda i: (i, 0))
    full = lambda s: pl.BlockSpec(s, lambda i: (0, 0))
    return pl.pallas_call(
        _mlp_body,
        grid=(nblk,),
        in_specs=[
            data_spec(EMB), data_spec(EMB), data_spec(EMB), data_spec(2),
            full((EMB, 128)), full((EMB, 128)), full((EMB, 128)),
            full((2, 128)), full((1, 128)),
            full((128, 64)), full((1, 64)),
            full((1, 64)), full((1, 1)),
        ],
        out_specs=pl.BlockSpec((BBLK,), lambda i: (i,)),
        out_shape=jax.ShapeDtypeStruct((HB,), jnp.float32),
    )(u, m, c, an, w1u, w1m, w1c, w1an, b1, w2, b2, w3, b3)


def kernel(user, movie, cluster, avg_rating, num_ratings,
           user_table, movie_table, cluster_table,
           W1, b1, W2, b2, W3, b3):
    user = user.astype(jnp.int32)
    movie = movie.astype(jnp.int32)
    cluster = cluster.astype(jnp.int32)

    u3 = user_table.reshape(NUM_USERS // 8, 8, EMB)
    m3 = movie_table.reshape(NUM_MOVIES // 8, 8, EMB)
    c3 = cluster_table.reshape(NUM_CLUSTERS // 8, 8, EMB)

    an = jnp.stack([avg_rating, num_ratings], axis=1)
    w1u, w1m = W1[:, :EMB].T, W1[:, EMB:2 * EMB].T
    w1c, w1an = W1[:, 2 * EMB:3 * EMB].T, W1[:, 3 * EMB:].T
    b1r, b2r, b3r = b1[None, :], b2[None, :], b3[None, :]

    embs = []
    for h in range(NSPLIT):
        sl = slice(h * HB, (h + 1) * HB)
        embs.append(_make_gather()(u3, m3, c3, user[sl], movie[sl],
                                   cluster[sl]))
    outs = []
    for h in range(NSPLIT):
        sl = slice(h * HB, (h + 1) * HB)
        u_emb, m_emb, c_emb = embs[h]
        outs.append(_mlp(u_emb, m_emb, c_emb, an[sl],
                         w1u, w1m, w1c, w1an, b1r, W2.T, b2r, W3, b3r))
    return jnp.concatenate(outs)


# speedup vs baseline: 1.0227x; 1.0227x over previous
import functools

import jax
import jax.numpy as jnp
from jax import lax
from jax.experimental import pallas as pl
from jax.experimental.pallas import tpu as pltpu
from jax.experimental.pallas import tpu_sc as plsc

NUM_USERS = 1000000
NUM_MOVIES = 100000
NUM_CLUSTERS = 1000
EMB = 64
BATCH = 16384

NC, NS = 2, 16
NW = NC * NS
BPW = BATCH // NW

BBLK = 2048


def _gather_body(ut, mt, ct, ui, mi, ci, ou, om, oc, idx_v, rows_v, sem):
    wid = lax.axis_index("s") * NC + lax.axis_index("c")
    base = wid * BPW
    for tbl, idxr, outr in ((ut, ui, ou), (mt, mi, om), (ct, ci, oc)):
        pltpu.sync_copy(idxr.at[pl.ds(base, BPW)], idx_v)

        def body(g, _):
            v = idx_v[pl.ds(g * 16, 16)]
            for l in range(16):
                idx = v[l]
                pltpu.async_copy(tbl.at[idx >> 3, idx & 7],
                                 rows_v.at[g * 16 + l], sem)
            return 0

        lax.fori_loop(0, BPW // 16, body, 0, unroll=4)
        pltpu.make_async_copy(outr.at[pl.ds(base, BPW)], rows_v, sem).wait()
        pltpu.sync_copy(rows_v, outr.at[pl.ds(base, BPW)])


@functools.lru_cache(maxsize=None)
def _make_gather():
    return pl.kernel(
        _gather_body,
        out_type=(
            jax.ShapeDtypeStruct((BATCH, EMB), jnp.float32),
            jax.ShapeDtypeStruct((BATCH, EMB), jnp.float32),
            jax.ShapeDtypeStruct((BATCH, EMB), jnp.float32),
        ),
        mesh=plsc.VectorSubcoreMesh(core_axis_name="c", subcore_axis_name="s",
                                    num_cores=NC, num_subcores=NS),
        scratch_types=[
            pltpu.VMEM((BPW,), jnp.int32),
            pltpu.VMEM((BPW, EMB), jnp.float32),
            pltpu.SemaphoreType.DMA,
        ],
        compiler_params=pltpu.CompilerParams(needs_layout_passes=False),
    )


def _mlp_body(u, m, c, an, w1u, w1m, w1c, w1an, b1, w2, b2, w3, b3, out):
    h = jnp.dot(u[:], w1u[:], preferred_element_type=jnp.float32)
    h = h + jnp.dot(m[:], w1m[:], preferred_element_type=jnp.float32)
    h = h + jnp.dot(c[:], w1c[:], preferred_element_type=jnp.float32)
    h = h + jnp.dot(an[:], w1an[:], preferred_element_type=jnp.float32)
    h = jnp.maximum(h + b1[:], 0.0)
    h2 = jnp.dot(h, w2[:], preferred_element_type=jnp.float32) + b2[:]
    h2 = jnp.maximum(h2, 0.0)
    out[:] = jnp.dot(h2, w3[:], preferred_element_type=jnp.float32) + b3[:]


def _mlp(u, m, c, an, w1u, w1m, w1c, w1an, b1, w2, b2, w3, b3):
    nblk = BATCH // BBLK
    data_spec = lambda d: pl.BlockSpec((BBLK, d), lambda i: (i, 0))
    full = lambda s: pl.BlockSpec(s, lambda i: (0, 0))
    return pl.pallas_call(
        _mlp_body,
        grid=(nblk,),
        in_specs=[
            data_spec(EMB), data_spec(EMB), data_spec(EMB), data_spec(2),
            full((EMB, 128)), full((EMB, 128)), full((EMB, 128)),
            full((2, 128)), full((1, 128)),
            full((128, 64)), full((1, 64)),
            full((64, 1)), full((1, 1)),
        ],
        out_specs=pl.BlockSpec((BBLK, 1), lambda i: (i, 0)),
        out_shape=jax.ShapeDtypeStruct((BATCH, 1), jnp.float32),
    )(u, m, c, an, w1u, w1m, w1c, w1an, b1, w2, b2, w3, b3)


def kernel(user, movie, cluster, avg_rating, num_ratings,
           user_table, movie_table, cluster_table,
           W1, b1, W2, b2, W3, b3):
    user = user.astype(jnp.int32)
    movie = movie.astype(jnp.int32)
    cluster = cluster.astype(jnp.int32)

    u3 = user_table.reshape(NUM_USERS // 8, 8, EMB)
    m3 = movie_table.reshape(NUM_MOVIES // 8, 8, EMB)
    c3 = cluster_table.reshape(NUM_CLUSTERS // 8, 8, EMB)
    u_emb, m_emb, c_emb = _make_gather()(u3, m3, c3, user, movie, cluster)

    an = jnp.stack([avg_rating, num_ratings], axis=1)
    out = _mlp(u_emb, m_emb, c_emb, an,
               W1[:, :EMB].T, W1[:, EMB:2 * EMB].T, W1[:, 2 * EMB:3 * EMB].T,
               W1[:, 3 * EMB:].T, b1[None, :],
               W2.T, b2[None, :], W3.T, b3[None, :])
    return out[:, 0]


# speedup vs baseline: 1.0292x; 1.0063x over previous
import functools

import jax
import jax.numpy as jnp
from jax import lax
from jax.experimental import pallas as pl
from jax.experimental.pallas import tpu as pltpu
from jax.experimental.pallas import tpu_sc as plsc

NUM_USERS = 1000000
NUM_MOVIES = 100000
NUM_CLUSTERS = 1000
EMB = 64
BATCH = 16384

NC, NS = 2, 16
NW = NC * NS
BPW = BATCH // NW

BBLK = 2048


def _gather_body(ut, mt, ct, ui, mi, ci, ou, om, oc, idx_v, rows_v, sem):
    wid = lax.axis_index("s") * NC + lax.axis_index("c")
    base = wid * BPW
    for tbl, idxr, outr in ((ut, ui, ou), (mt, mi, om), (ct, ci, oc)):
        pltpu.sync_copy(idxr.at[pl.ds(base, BPW)], idx_v)

        def body(g, _):
            v = idx_v[pl.ds(g * 16, 16)]
            for l in range(16):
                idx = v[l]
                pltpu.async_copy(tbl.at[idx >> 3, idx & 7],
                                 rows_v.at[g * 16 + l], sem)
            return 0

        lax.fori_loop(0, BPW // 16, body, 0)
        pltpu.make_async_copy(outr.at[pl.ds(base, BPW)], rows_v, sem).wait()
        pltpu.sync_copy(rows_v, outr.at[pl.ds(base, BPW)])


@functools.lru_cache(maxsize=None)
def _make_gather():
    return pl.kernel(
        _gather_body,
        out_type=(
            jax.ShapeDtypeStruct((BATCH, EMB), jnp.float32),
            jax.ShapeDtypeStruct((BATCH, EMB), jnp.float32),
            jax.ShapeDtypeStruct((BATCH, EMB), jnp.float32),
        ),
        mesh=plsc.VectorSubcoreMesh(core_axis_name="c", subcore_axis_name="s",
                                    num_cores=NC, num_subcores=NS),
        scratch_types=[
            pltpu.VMEM((BPW,), jnp.int32),
            pltpu.VMEM((BPW, EMB), jnp.float32),
            pltpu.SemaphoreType.DMA,
        ],
        compiler_params=pltpu.CompilerParams(needs_layout_passes=False),
    )


def _mlp_body(u, m, c, an, w1u, w1m, w1c, w1an, b1, w2, b2, w3, b3, out):
    h = jnp.dot(u[:], w1u[:], preferred_element_type=jnp.float32)
    h = h + jnp.dot(m[:], w1m[:], preferred_element_type=jnp.float32)
    h = h + jnp.dot(c[:], w1c[:], preferred_element_type=jnp.float32)
    h = h + jnp.dot(an[:], w1an[:], preferred_element_type=jnp.float32)
    h = jnp.maximum(h + b1[:], 0.0)
    h2 = jnp.dot(h, w2[:], preferred_element_type=jnp.float32) + b2[:]
    h2 = jnp.maximum(h2, 0.0)
    out[:] = jnp.dot(h2, w3[:], preferred_element_type=jnp.float32) + b3[:]


def _mlp(u, m, c, an, w1u, w1m, w1c, w1an, b1, w2, b2, w3, b3):
    nblk = BATCH // BBLK
    data_spec = lambda d: pl.BlockSpec((BBLK, d), lambda i: (i, 0))
    full = lambda s: pl.BlockSpec(s, lambda i: (0, 0))
    return pl.pallas_call(
        _mlp_body,
        grid=(nblk,),
        in_specs=[
            data_spec(EMB), data_spec(EMB), data_spec(EMB), data_spec(2),
            full((EMB, 128)), full((EMB, 128)), full((EMB, 128)),
            full((2, 128)), full((1, 128)),
            full((128, 64)), full((1, 64)),
            full((64, 1)), full((1, 1)),
        ],
        out_specs=pl.BlockSpec((BBLK, 1), lambda i: (i, 0)),
        out_shape=jax.ShapeDtypeStruct((BATCH, 1), jnp.float32),
    )(u, m, c, an, w1u, w1m, w1c, w1an, b1, w2, b2, w3, b3)


def kernel(user, movie, cluster, avg_rating, num_ratings,
           user_table, movie_table, cluster_table,
           W1, b1, W2, b2, W3, b3):
    user = user.astype(jnp.int32)
    movie = movie.astype(jnp.int32)
    cluster = cluster.astype(jnp.int32)

    u3 = user_table.reshape(NUM_USERS // 8, 8, EMB)
    m3 = movie_table.reshape(NUM_MOVIES // 8, 8, EMB)
    c3 = cluster_table.reshape(NUM_CLUSTERS // 8, 8, EMB)
    u_emb, m_emb, c_emb = _make_gather()(u3, m3, c3, user, movie, cluster)

    an = jnp.stack([avg_rating, num_ratings], axis=1)
    out = _mlp(u_emb, m_emb, c_emb, an,
               W1[:, :EMB].T, W1[:, EMB:2 * EMB].T, W1[:, 2 * EMB:3 * EMB].T,
               W1[:, 3 * EMB:].T, b1[None, :],
               W2.T, b2[None, :], W3.T, b3[None, :])
    return out[:, 0]
